# Initial kernel scaffold; baseline (speedup 1.0000x reference)
import jax
import jax.numpy as jnp
from jax.experimental import pallas as pl


def kernel(positive_sample, negative_sample, entity_embedding, relation_embedding, entity_cov, relation_cov):
    raise NotImplementedError("write your pallas kernel here")


# speedup vs baseline: 2.1269x; 2.1269x over previous
import functools

import jax
import jax.numpy as jnp
from jax import lax
from jax.experimental import pallas as pl
from jax.experimental.pallas import tpu as pltpu
from jax.experimental.pallas import tpu_sc as plsc

_GAMMA = 12.0
_B = 1024
_NEG = 200
_NEG_PAD = 208
_D = 128
_NC = 2
_NS = 16
_NW = _NC * _NS
_BPW = _B // _NW
_NCHUNK = _D // 16
_NGROUP = _NEG_PAD // 16


def _sc_score(head_idx, rel_idx, neg_idx, entity_embedding, relation_embedding):
    mesh = plsc.VectorSubcoreMesh(core_axis_name="c", subcore_axis_name="s")

    @functools.partial(
        pl.kernel,
        out_type=jax.ShapeDtypeStruct((_B, _NEG_PAD), jnp.float32),
        mesh=mesh,
        scratch_types=[
            pltpu.VMEM((_BPW,), jnp.int32),
            pltpu.VMEM((_BPW,), jnp.int32),
            pltpu.VMEM((_BPW, _D), jnp.float32),
            pltpu.VMEM((_BPW, _D), jnp.float32),
            pltpu.VMEM((_BPW, 2, _NEG_PAD // 2), jnp.int32),
            pltpu.VMEM((_NEG_PAD, _D), jnp.float32),
            pltpu.VMEM((_NEG_PAD,), jnp.float32),
            pltpu.SemaphoreType.DMA,
        ],
    )
    def k(head_idx_hbm, rel_idx_hbm, neg_hbm, ent_hbm, rel_emb_hbm, out_hbm,
          hidx_v, ridx_v, head_rows, rel_rows, neg_v, tails, score_row, sem):
        wid = lax.axis_index("s") * _NC + lax.axis_index("c")
        base = wid * _BPW

        pltpu.sync_copy(head_idx_hbm.at[pl.ds(base, _BPW)], hidx_v)
        pltpu.sync_copy(rel_idx_hbm.at[pl.ds(base, _BPW)], ridx_v)
        pltpu.sync_copy(neg_hbm.at[pl.ds(base, _BPW)], neg_v)
        pltpu.async_copy(ent_hbm.at[hidx_v], head_rows, sem).wait()
        pltpu.async_copy(rel_emb_hbm.at[ridx_v], rel_rows, sem).wait()

        half = _NEG_PAD // 2
        iota16 = lax.iota(jnp.int32, 16)
        perms = [iota16 ^ k for k in (1, 2, 4, 8)]
        lane_eq = [iota16 == n for n in range(16)]

        def body_b(b, carry):
            h1 = pltpu.async_copy(
                ent_hbm.at[neg_v.at[b, 0]], tails.at[pl.ds(0, half)], sem)
            h2 = pltpu.async_copy(
                ent_hbm.at[neg_v.at[b, 1]], tails.at[pl.ds(half, half)], sem)
            h1.wait()
            h2.wait()

            qs = [head_rows[b, pl.ds(c * 16, 16)] + rel_rows[b, pl.ds(c * 16, 16)]
                  for c in range(_NCHUNK)]

            def body_g(g, carry2):
                vec = jnp.zeros((16,), jnp.float32)
                for n in range(16):
                    row = g * 16 + n
                    acc = jnp.abs(qs[0] - tails[row, pl.ds(0, 16)])
                    for c in range(1, _NCHUNK):
                        acc = acc + jnp.abs(qs[c] - tails[row, pl.ds(c * 16, 16)])
                    for p in perms:
                        acc = acc + acc.at[p].get(mode="promise_in_bounds")
                    vec = jnp.where(lane_eq[n], acc, vec)
                score_row[pl.ds(g * 16, 16)] = _GAMMA - vec
                return carry2

            lax.fori_loop(0, _NGROUP, body_g, 0)
            pltpu.sync_copy(score_row, out_hbm.at[base + b])
            return carry

        lax.fori_loop(0, _BPW, body_b, 0)

    return k(head_idx, rel_idx, neg_idx, entity_embedding, relation_embedding)


def kernel(positive_sample, negative_sample, entity_embedding,
           relation_embedding, entity_cov, relation_cov):
    del entity_cov, relation_cov
    head_idx = positive_sample[:, 0].astype(jnp.int32)
    rel_idx = positive_sample[:, 1].astype(jnp.int32)
    neg_pad = jnp.concatenate(
        [negative_sample.astype(jnp.int32),
         jnp.zeros((_B, _NEG_PAD - _NEG), jnp.int32)], axis=1)
    neg_idx = neg_pad.reshape(_B, 2, _NEG_PAD // 2)
    out = _sc_score(head_idx, rel_idx, neg_idx, entity_embedding,
                    relation_embedding)
    return out[:, :_NEG]
